# baseline (speedup 1.0000x reference)
import functools

import jax
import jax.numpy as jnp
from jax import lax
from jax.experimental import pallas as pl
from jax.experimental.pallas import tpu as pltpu
from jax.experimental.pallas import tpu_sc as plsc

D = 64
L = 16
NC = 2
NS = 16
CH = 128


@functools.cache
def _make_kernel(B):
    NW = NC * NS
    BPW = B // NW
    NCH = BPW // CH
    mesh = plsc.VectorSubcoreMesh(core_axis_name="c", subcore_axis_name="s")

    @functools.partial(
        pl.kernel,
        mesh=mesh,
        out_type=jax.ShapeDtypeStruct((B,), jnp.float32),
        compiler_params=pltpu.CompilerParams(
            needs_layout_passes=False, use_tc_tiling_on_sc=False),
        scratch_types=[
            pltpu.VMEM((NCH, CH), jnp.int32),
            pltpu.VMEM((NCH, CH), jnp.int32),
            pltpu.VMEM((BPW, D), jnp.float32),
            pltpu.VMEM((BPW, D), jnp.float32),
            pltpu.VMEM((BPW,), jnp.float32),
            pltpu.SemaphoreType.DMA,
        ],
    )
    def kern(uids_hbm, iids_hbm, utab_hbm, itab_hbm, out_hbm,
             uids_v, iids_v, urows_v, irows_v, out_v, sem):
        wid = lax.axis_index("s") * NC + lax.axis_index("c")
        base = wid * BPW

        for c in range(NCH):
            pltpu.sync_copy(uids_hbm.at[pl.ds(base + c * CH, CH)], uids_v.at[c])
            pltpu.sync_copy(iids_hbm.at[pl.ds(base + c * CH, CH)], iids_v.at[c])

        copies = []
        for c in range(NCH):
            copies.append(pltpu.async_copy(
                utab_hbm.at[uids_v.at[c]], urows_v.at[pl.ds(c * CH, CH)], sem))
            copies.append(pltpu.async_copy(
                itab_hbm.at[iids_v.at[c]], irows_v.at[pl.ds(c * CH, CH)], sem))
        for cp in copies:
            cp.wait()

        lanes = lax.iota(jnp.int32, L)

        def body(it, carry):
            r0 = it * L
            res = jnp.zeros((L,), jnp.float32)
            for k in range(L):
                r = r0 + k
                acc = jnp.zeros((L,), jnp.float32)
                for j in range(D // L):
                    pu = urows_v[r, pl.ds(j * L, L)]
                    pi = irows_v[r, pl.ds(j * L, L)]
                    acc = acc + pu * pi
                res = jnp.where(lanes == k, jnp.sum(acc), res)
            out_v[pl.ds(r0, L)] = res
            return carry

        lax.fori_loop(0, BPW // L, body, 0)

        pltpu.sync_copy(out_v, out_hbm.at[pl.ds(base, BPW)])

    return kern


def kernel(user_ids, item_ids, user_table, item_table):
    B = user_ids.shape[0]
    kern = _make_kernel(B)
    return kern(user_ids.astype(jnp.int32), item_ids.astype(jnp.int32),
                user_table, item_table)


# speedup vs baseline: 1.5013x; 1.5013x over previous
import functools

import jax
import jax.numpy as jnp
from jax import lax
from jax.experimental import pallas as pl
from jax.experimental.pallas import tpu as pltpu
from jax.experimental.pallas import tpu_sc as plsc

D = 64
L = 16
NC = 2
NS = 16
NW = NC * NS
CW = 512
NCHUNK = 999936 // CW
TAIL = 999936
RING = 64
RP = 128


@functools.cache
def _make_gather_kernel(B, V):
    mesh = plsc.VectorSubcoreMesh(core_axis_name="c", subcore_axis_name="s")
    NV = B // L
    KPW = (NCHUNK + NW - 1) // NW + 1

    @functools.partial(
        pl.kernel,
        mesh=mesh,
        out_type=(jax.ShapeDtypeStruct((B * RP,), jnp.float32),
                  jax.ShapeDtypeStruct((B * RP,), jnp.float32)),
        compiler_params=pltpu.CompilerParams(
            needs_layout_passes=False, use_tc_tiling_on_sc=True),
        scratch_types=[
            pltpu.VMEM((B,), jnp.int32),
            pltpu.VMEM((B + L,), jnp.int32),
            pltpu.VMEM((B + L,), jnp.int32),
            pltpu.VMEM((D, CW), jnp.float32),
            pltpu.VMEM((RING * RP,), jnp.float32),
            pltpu.SemaphoreType.DMA,
            pltpu.SemaphoreType.DMA,
        ],
    )
    def kern(uids_hbm, iids_hbm, utT_hbm, itT_hbm, ustage_hbm, istage_hbm,
             ids_v, lst_id, lst_b, buf_v, ring_v, semc, semr):
        wid = lax.axis_index("s") * NC + lax.axis_index("c")
        lanes = lax.iota(jnp.int32, L)

        def one_table(ids_hbm, tT_hbm, stage_hbm):
            pltpu.sync_copy(ids_hbm, ids_v)

            def scan_ids(j, off):
                idv = ids_v[pl.ds(j * L, L)]
                bv = jnp.full((L,), j * L, jnp.int32) + lanes
                keep = ((idv >> 9) & (NW - 1)) == wid
                cnt = plsc.all_reduce_population_count(keep)[0]
                plsc.store_compressed(lst_id.at[pl.ds(off, L)], idv, mask=keep)
                plsc.store_compressed(lst_b.at[pl.ds(off, L)], bv, mask=keep)
                return off + cnt

            n = lax.fori_loop(0, NV, scan_ids, 0)
            nv = (n + L - 1) // L

            def chunk_body(k, pending):
                g = wid + k * NW

                def serve(pending):
                    cps = []
                    for i in range(D // 8):
                        cps.append(pltpu.async_copy(
                            tT_hbm.at[pl.ds(i * 8, 8), pl.ds(g * CW, CW)],
                            buf_v.at[pl.ds(i * 8, 8), :], semc))
                    for cp in cps:
                        cp.wait()
                    col0 = g * CW

                    def scan_list(j, pending):
                        idv = lst_id[pl.ds(j * L, L)]
                        bv = lst_b[pl.ds(j * L, L)]
                        valid = (lanes + j * L) < n
                        hit = ((idv >> 9) == g) & valid
                        hit32 = hit.astype(jnp.int32)
                        nhit = plsc.all_reduce_population_count(hit)[0]

                        def lanes_body():
                            pend = pending
                            for lane in range(L):
                                mbit = hit32[lane]

                                @pl.when((mbit == 1) & (pend == RING))
                                def _():
                                    def drain1(_i, c):
                                        pltpu.make_async_copy(
                                            stage_hbm.at[pl.ds(0, RP)],
                                            ring_v.at[pl.ds(0, RP)],
                                            semr).wait()
                                        return c

                                    lax.fori_loop(0, RING, drain1, 0)

                                slot = jnp.where(pend == RING, 0, pend)

                                @pl.when(mbit == 1)
                                def _():
                                    q = idv[lane] - col0
                                    b = bv[lane]
                                    qs = jnp.full((L,), q, jnp.int32)
                                    for jj in range(D // L):
                                        dv = lanes + jj * L
                                        vals = plsc.load_gather(
                                            buf_v, [dv, qs])
                                        ring_v[pl.ds(slot * RP + jj * L, L)] = vals
                                    pltpu.async_copy(
                                        ring_v.at[pl.ds(slot * RP, RP)],
                                        stage_hbm.at[pl.ds(b * RP, RP)],
                                        semr)

                                pend = jnp.where(
                                    mbit == 1,
                                    jnp.where(pend == RING, 1, pend + 1),
                                    pend)
                            return pend

                        return lax.cond(nhit > 0, lanes_body, lambda: pending)

                    return lax.fori_loop(0, nv, scan_list, pending)

                return lax.cond(g < NCHUNK, serve, lambda p: p, pending)

            pending = lax.fori_loop(0, KPW, chunk_body, 0)

            def final_drain(_i, carry):
                pltpu.make_async_copy(
                    stage_hbm.at[pl.ds(0, RP)], ring_v.at[pl.ds(0, RP)],
                    semr).wait()
                return carry

            lax.fori_loop(0, pending, final_drain, 0)

        one_table(uids_hbm, utT_hbm, ustage_hbm)
        one_table(iids_hbm, itT_hbm, istage_hbm)

    return kern


@functools.cache
def _make_dot_kernel(B):
    mesh = plsc.VectorSubcoreMesh(core_axis_name="c", subcore_axis_name="s")
    BPW = B // NW
    GRP = 256

    @functools.partial(
        pl.kernel,
        mesh=mesh,
        out_type=jax.ShapeDtypeStruct((B,), jnp.float32),
        compiler_params=pltpu.CompilerParams(
            needs_layout_passes=False, use_tc_tiling_on_sc=False),
        scratch_types=[
            pltpu.VMEM((GRP * RP,), jnp.float32),
            pltpu.VMEM((GRP * RP,), jnp.float32),
            pltpu.VMEM((BPW,), jnp.int32),
            pltpu.VMEM((BPW,), jnp.int32),
            pltpu.VMEM((D * D,), jnp.float32),
            pltpu.VMEM((D * D,), jnp.float32),
            pltpu.VMEM((BPW,), jnp.float32),
        ],
    )
    def kern(uids_hbm, iids_hbm, ustage_hbm, istage_hbm, utail_hbm, itail_hbm,
             out_hbm, uv, iv, uidv, iidv, utl, itl, out_v):
        wid = lax.axis_index("s") * NC + lax.axis_index("c")
        base = wid * BPW
        lanes = lax.iota(jnp.int32, L)

        pltpu.sync_copy(uids_hbm.at[pl.ds(base, BPW)], uidv)
        pltpu.sync_copy(iids_hbm.at[pl.ds(base, BPW)], iidv)
        pltpu.sync_copy(utail_hbm, utl)
        pltpu.sync_copy(itail_hbm, itl)

        for grp in range(BPW // GRP):
            g0 = base + grp * GRP
            pltpu.sync_copy(ustage_hbm.at[pl.ds(g0 * RP, GRP * RP)], uv)
            pltpu.sync_copy(istage_hbm.at[pl.ds(g0 * RP, GRP * RP)], iv)

            def body(it, carry):
                r0 = it * L
                res = jnp.zeros((L,), jnp.float32)
                for kk in range(L):
                    r = r0 + kk
                    acc = jnp.zeros((L,), jnp.float32)
                    for j in range(D // L):
                        pu = uv[pl.ds(r * RP + j * L, L)]
                        pi = iv[pl.ds(r * RP + j * L, L)]
                        acc = acc + pu * pi
                    res = jnp.where(lanes == kk, jnp.sum(acc), res)

                ob = grp * GRP + r0
                uidvec = uidv[pl.ds(ob, L)]
                iidvec = iidv[pl.ds(ob, L)]
                tl = (uidvec >= TAIL) | (iidvec >= TAIL)
                tl32 = tl.astype(jnp.int32)
                nt = plsc.all_reduce_population_count(tl)[0]

                @pl.when(nt > 0)
                def _():
                    fixed = res
                    for lane in range(L):
                        @pl.when(tl32[lane] == 1)
                        def _():
                            r = r0 + lane
                            uid = uidvec[lane]
                            iid = iidvec[lane]
                            uo = lax.max(uid - TAIL, 0) * D
                            io = lax.max(iid - TAIL, 0) * D
                            acc2 = jnp.zeros((L,), jnp.float32)
                            for j in range(D // L):
                                us = uv[pl.ds(r * RP + j * L, L)]
                                ut = utl[pl.ds(uo + j * L, L)]
                                uu = jnp.where(uid >= TAIL, ut, us)
                                ss = iv[pl.ds(r * RP + j * L, L)]
                                st = itl[pl.ds(io + j * L, L)]
                                ii = jnp.where(iid >= TAIL, st, ss)
                                acc2 = acc2 + uu * ii
                            s = jnp.sum(acc2)
                            cur = out_v[pl.ds(ob, L)]
                            out_v[pl.ds(ob, L)] = jnp.where(
                                lanes == lane, s, cur)

                    out_v[pl.ds(ob, L)] = jnp.where(
                        tl, out_v[pl.ds(ob, L)], fixed)

                @pl.when(nt == 0)
                def _():
                    out_v[pl.ds(ob, L)] = res

                return carry

            lax.fori_loop(0, GRP // L, body, 0)

        pltpu.sync_copy(out_v, out_hbm.at[pl.ds(base, BPW)])

    return kern


def kernel(user_ids, item_ids, user_table, item_table):
    B = user_ids.shape[0]
    V = user_table.shape[0]
    uids = user_ids.astype(jnp.int32)
    iids = item_ids.astype(jnp.int32)
    utail = user_table[TAIL:, :].reshape(-1)
    itail = item_table[TAIL:, :].reshape(-1)
    gather = _make_gather_kernel(B, V)
    ustage, istage = gather(uids, iids, user_table.T, item_table.T)
    dot = _make_dot_kernel(B)
    return dot(uids, iids, ustage, istage, utail, itail)


# speedup vs baseline: 2.0005x; 1.3325x over previous
import functools

import jax
import jax.numpy as jnp
from jax import lax
from jax.experimental import pallas as pl
from jax.experimental.pallas import tpu as pltpu
from jax.experimental.pallas import tpu_sc as plsc

D = 64
L = 16
NC = 2
NS = 16
NW = NC * NS
CW = 512
NCHUNK = 999936 // CW
TAIL = 999936
RING = 64
RP = 128


@functools.cache
def _make_gather_kernel(B, V):
    mesh = plsc.VectorSubcoreMesh(core_axis_name="c", subcore_axis_name="s")
    NV = B // L
    KPW = (NCHUNK + NW - 1) // NW + 1

    @functools.partial(
        pl.kernel,
        mesh=mesh,
        out_type=(jax.ShapeDtypeStruct((B * RP,), jnp.float32),
                  jax.ShapeDtypeStruct((B * RP,), jnp.float32)),
        compiler_params=pltpu.CompilerParams(
            needs_layout_passes=False, use_tc_tiling_on_sc=True),
        scratch_types=[
            pltpu.VMEM((B,), jnp.int32),
            pltpu.VMEM((B + L,), jnp.int32),
            pltpu.VMEM((B + L,), jnp.int32),
            pltpu.VMEM((2 * D, CW), jnp.float32),
            pltpu.VMEM((RING * RP,), jnp.float32),
            pltpu.SemaphoreType.DMA,
            pltpu.SemaphoreType.DMA,
            pltpu.SemaphoreType.DMA,
        ],
    )
    def kern(uids_hbm, iids_hbm, utT_hbm, itT_hbm, ustage_hbm, istage_hbm,
             ids_v, lst_id, lst_b, buf_v, ring_v, semc0, semc1, semr):
        wid = lax.axis_index("s") * NC + lax.axis_index("c")
        lanes = lax.iota(jnp.int32, L)

        def one_table(ids_hbm, tT_hbm, stage_hbm):
            pltpu.sync_copy(ids_hbm, ids_v)

            def scan_ids(j, off):
                idv = ids_v[pl.ds(j * L, L)]
                bv = jnp.full((L,), j * L, jnp.int32) + lanes
                keep = ((idv >> 9) & (NW - 1)) == wid
                cnt = plsc.all_reduce_population_count(keep)[0]
                plsc.store_compressed(lst_id.at[pl.ds(off, L)], idv, mask=keep)
                plsc.store_compressed(lst_b.at[pl.ds(off, L)], bv, mask=keep)
                return off + cnt

            n = lax.fori_loop(0, NV, scan_ids, 0)
            nv = (n + L - 1) // L

            def issue(k, sem, boff):
                g = wid + k * NW

                @pl.when(g < NCHUNK)
                def _():
                    for i in range(D // 8):
                        pltpu.async_copy(
                            tT_hbm.at[pl.ds(i * 8, 8), pl.ds(g * CW, CW)],
                            buf_v.at[pl.ds(boff + i * 8, 8), :], sem)

            def wait_chunk(k, sem, boff):
                g = wid + k * NW

                @pl.when(g < NCHUNK)
                def _():
                    for i in range(D // 8):
                        pltpu.make_async_copy(
                            tT_hbm.at[pl.ds(i * 8, 8), pl.ds(g * CW, CW)],
                            buf_v.at[pl.ds(boff + i * 8, 8), :], sem).wait()

            def process(k, boff, pending):
                g = wid + k * NW

                def serve(pending):
                    col0 = g * CW

                    def scan_list(j, pending):
                        idv = lst_id[pl.ds(j * L, L)]
                        bv = lst_b[pl.ds(j * L, L)]
                        valid = (lanes + j * L) < n
                        hit = ((idv >> 9) == g) & valid
                        hit32 = hit.astype(jnp.int32)
                        nhit = plsc.all_reduce_population_count(hit)[0]

                        def lanes_body():
                            pend = pending
                            for lane in range(L):
                                mbit = hit32[lane]

                                @pl.when((mbit == 1) & (pend == RING))
                                def _():
                                    def drain1(_i, c):
                                        pltpu.make_async_copy(
                                            stage_hbm.at[pl.ds(0, RP)],
                                            ring_v.at[pl.ds(0, RP)],
                                            semr).wait()
                                        return c

                                    lax.fori_loop(0, RING, drain1, 0)

                                slot = jnp.where(pend == RING, 0, pend)

                                @pl.when(mbit == 1)
                                def _():
                                    q = idv[lane] - col0
                                    b = bv[lane]
                                    qs = jnp.full((L,), q, jnp.int32)
                                    dbase = jnp.full((L,), boff, jnp.int32)
                                    for jj in range(D // L):
                                        dv = dbase + lanes + jj * L
                                        vals = plsc.load_gather(
                                            buf_v, [dv, qs])
                                        ring_v[pl.ds(slot * RP + jj * L, L)] = vals
                                    pltpu.async_copy(
                                        ring_v.at[pl.ds(slot * RP, RP)],
                                        stage_hbm.at[pl.ds(b * RP, RP)],
                                        semr)

                                pend = jnp.where(
                                    mbit == 1,
                                    jnp.where(pend == RING, 1, pend + 1),
                                    pend)
                            return pend

                        return lax.cond(nhit > 0, lanes_body, lambda: pending)

                    return lax.fori_loop(0, nv, scan_list, pending)

                return lax.cond(g < NCHUNK, serve, lambda p: p, pending)

            issue(0, semc0, 0)

            def pair(kk, pending):
                k0 = kk * 2
                wait_chunk(k0, semc0, 0)
                issue(k0 + 1, semc1, D)
                pending = process(k0, 0, pending)
                wait_chunk(k0 + 1, semc1, D)
                issue(k0 + 2, semc0, 0)
                pending = process(k0 + 1, D, pending)
                return pending

            pending = lax.fori_loop(0, (KPW + 1) // 2, pair, 0)

            def final_drain(_i, carry):
                pltpu.make_async_copy(
                    stage_hbm.at[pl.ds(0, RP)], ring_v.at[pl.ds(0, RP)],
                    semr).wait()
                return carry

            lax.fori_loop(0, pending, final_drain, 0)

        one_table(uids_hbm, utT_hbm, ustage_hbm)
        one_table(iids_hbm, itT_hbm, istage_hbm)

    return kern


@functools.cache
def _make_dot_kernel(B):
    mesh = plsc.VectorSubcoreMesh(core_axis_name="c", subcore_axis_name="s")
    BPW = B // NW
    GRP = 256

    @functools.partial(
        pl.kernel,
        mesh=mesh,
        out_type=jax.ShapeDtypeStruct((B,), jnp.float32),
        compiler_params=pltpu.CompilerParams(
            needs_layout_passes=False, use_tc_tiling_on_sc=False),
        scratch_types=[
            pltpu.VMEM((GRP * RP,), jnp.float32),
            pltpu.VMEM((GRP * RP,), jnp.float32),
            pltpu.VMEM((BPW,), jnp.int32),
            pltpu.VMEM((BPW,), jnp.int32),
            pltpu.VMEM((D * D,), jnp.float32),
            pltpu.VMEM((D * D,), jnp.float32),
            pltpu.VMEM((BPW,), jnp.float32),
        ],
    )
    def kern(uids_hbm, iids_hbm, ustage_hbm, istage_hbm, utail_hbm, itail_hbm,
             out_hbm, uv, iv, uidv, iidv, utl, itl, out_v):
        wid = lax.axis_index("s") * NC + lax.axis_index("c")
        base = wid * BPW
        lanes = lax.iota(jnp.int32, L)

        pltpu.sync_copy(uids_hbm.at[pl.ds(base, BPW)], uidv)
        pltpu.sync_copy(iids_hbm.at[pl.ds(base, BPW)], iidv)
        pltpu.sync_copy(utail_hbm, utl)
        pltpu.sync_copy(itail_hbm, itl)

        for grp in range(BPW // GRP):
            g0 = base + grp * GRP
            pltpu.sync_copy(ustage_hbm.at[pl.ds(g0 * RP, GRP * RP)], uv)
            pltpu.sync_copy(istage_hbm.at[pl.ds(g0 * RP, GRP * RP)], iv)

            def body(it, carry):
                r0 = it * L
                res = jnp.zeros((L,), jnp.float32)
                for kk in range(L):
                    r = r0 + kk
                    acc = jnp.zeros((L,), jnp.float32)
                    for j in range(D // L):
                        pu = uv[pl.ds(r * RP + j * L, L)]
                        pi = iv[pl.ds(r * RP + j * L, L)]
                        acc = acc + pu * pi
                    res = jnp.where(lanes == kk, jnp.sum(acc), res)

                ob = grp * GRP + r0
                uidvec = uidv[pl.ds(ob, L)]
                iidvec = iidv[pl.ds(ob, L)]
                tl = (uidvec >= TAIL) | (iidvec >= TAIL)
                tl32 = tl.astype(jnp.int32)
                nt = plsc.all_reduce_population_count(tl)[0]

                @pl.when(nt > 0)
                def _():
                    fixed = res
                    for lane in range(L):
                        @pl.when(tl32[lane] == 1)
                        def _():
                            r = r0 + lane
                            uid = uidvec[lane]
                            iid = iidvec[lane]
                            uo = lax.max(uid - TAIL, 0) * D
                            io = lax.max(iid - TAIL, 0) * D
                            acc2 = jnp.zeros((L,), jnp.float32)
                            for j in range(D // L):
                                us = uv[pl.ds(r * RP + j * L, L)]
                                ut = utl[pl.ds(uo + j * L, L)]
                                uu = jnp.where(uid >= TAIL, ut, us)
                                ss = iv[pl.ds(r * RP + j * L, L)]
                                st = itl[pl.ds(io + j * L, L)]
                                ii = jnp.where(iid >= TAIL, st, ss)
                                acc2 = acc2 + uu * ii
                            s = jnp.sum(acc2)
                            cur = out_v[pl.ds(ob, L)]
                            out_v[pl.ds(ob, L)] = jnp.where(
                                lanes == lane, s, cur)

                    out_v[pl.ds(ob, L)] = jnp.where(
                        tl, out_v[pl.ds(ob, L)], fixed)

                @pl.when(nt == 0)
                def _():
                    out_v[pl.ds(ob, L)] = res

                return carry

            lax.fori_loop(0, GRP // L, body, 0)

        pltpu.sync_copy(out_v, out_hbm.at[pl.ds(base, BPW)])

    return kern


def kernel(user_ids, item_ids, user_table, item_table):
    B = user_ids.shape[0]
    V = user_table.shape[0]
    uids = user_ids.astype(jnp.int32)
    iids = item_ids.astype(jnp.int32)
    utail = user_table[TAIL:, :].reshape(-1)
    itail = item_table[TAIL:, :].reshape(-1)
    gather = _make_gather_kernel(B, V)
    ustage, istage = gather(uids, iids, user_table.T, item_table.T)
    dot = _make_dot_kernel(B)
    return dot(uids, iids, ustage, istage, utail, itail)


# speedup vs baseline: 2.1278x; 1.0636x over previous
import functools

import jax
import jax.numpy as jnp
from jax import lax
from jax.experimental import pallas as pl
from jax.experimental.pallas import tpu as pltpu
from jax.experimental.pallas import tpu_sc as plsc

D = 64
L = 16
NC = 2
NS = 16
NW = NC * NS
CW = 512
NCHUNK = 999936 // CW
TAIL = 999936
RING = 64
RP = 128


NQ = 248
NR = 8


@functools.cache
def _make_gather_kernel(B, V):
    mesh = plsc.VectorSubcoreMesh(core_axis_name="c", subcore_axis_name="s")
    NV = B // L

    @functools.partial(
        pl.kernel,
        mesh=mesh,
        out_type=(jax.ShapeDtypeStruct((B * RP,), jnp.float32),
                  jax.ShapeDtypeStruct((B * RP,), jnp.float32)),
        compiler_params=pltpu.CompilerParams(
            needs_layout_passes=False, use_tc_tiling_on_sc=True),
        scratch_types=[
            pltpu.VMEM((B,), jnp.int32),
            pltpu.VMEM((B + L,), jnp.int32),
            pltpu.VMEM((B + L,), jnp.int32),
            pltpu.VMEM((NR * D, 128), jnp.float32),
            pltpu.VMEM((RING * RP,), jnp.float32),
            pltpu.SMEM((256,), jnp.int32),
            pltpu.SMEM((256,), jnp.int32),
        ] + [pltpu.SemaphoreType.DMA] * (NR + 1),
    )
    def kern(uids_hbm, iids_hbm, utT_hbm, itT_hbm, ustage_hbm, istage_hbm,
             ids_v, lst, lst2, qbuf, ring_v, counts_s, pos_s, *sems):
        semq = sems[:NR]
        semr = sems[NR]
        wid = lax.axis_index("s") * NC + lax.axis_index("c")
        lanes = lax.iota(jnp.int32, L)

        def one_table(ids_hbm, tT_hbm, stage_hbm):
            pltpu.sync_copy(ids_hbm, ids_v)

            def scan_ids(j, off):
                idv = ids_v[pl.ds(j * L, L)]
                bv = jnp.full((L,), j * L, jnp.int32) + lanes
                keep = ((idv >> 9) & (NW - 1)) == wid
                cnt = plsc.all_reduce_population_count(keep)[0]
                packed = (bv << 15) | ((idv >> 14) << 9) | (idv & 0x1FF)
                plsc.store_compressed(lst.at[pl.ds(off, L)], packed, mask=keep)
                return off + cnt

            n = lax.fori_loop(0, NV, scan_ids, 0)
            nv = (n + L - 1) // L

            def zero(i, c):
                counts_s[i] = 0
                return c

            lax.fori_loop(0, 256, zero, 0)

            def hist(j, c):
                vv = lst[pl.ds(j * L, L)]
                for lane in range(L):
                    kk = (vv[lane] >> 7) & 0xFF
                    inc = jnp.where(j * L + lane < n, 1, 0)
                    counts_s[kk] = counts_s[kk] + inc
                return c

            lax.fori_loop(0, nv, hist, 0)

            def pref(k, run):
                pos_s[k] = run
                return run + counts_s[k]

            lax.fori_loop(0, 256, pref, 0)

            def scat(j, c):
                vv = lst[pl.ds(j * L, L)]
                for lane in range(L):
                    kk = (vv[lane] >> 7) & 0xFF
                    p = pos_s[kk]
                    ok = j * L + lane < n
                    onehot = (lanes == lane) & ok
                    plsc.store_scatter(
                        lst2, [jnp.full((L,), p, jnp.int32)], vv, mask=onehot)
                    pos_s[kk] = p + jnp.where(ok, 1, 0)
                return c

            lax.fori_loop(0, nv, scat, 0)

            def issue(Q, sem, boff):
                g = wid + (Q >> 2) * NW
                col = (g * 4 + (Q & 3)) * 128

                @pl.when(g < NCHUNK)
                def _():
                    pltpu.async_copy(
                        tT_hbm.at[:, pl.ds(col, 128)],
                        qbuf.at[pl.ds(boff, D), :], sem)

            def wait_q(Q, sem, boff):
                g = wid + (Q >> 2) * NW
                col = (g * 4 + (Q & 3)) * 128

                @pl.when(g < NCHUNK)
                def _():
                    pltpu.make_async_copy(
                        tT_hbm.at[:, pl.ds(col, 128)],
                        qbuf.at[pl.ds(boff, D), :], sem).wait()

            def process(Q, boff, pending):
                g = wid + (Q >> 2) * NW
                cnt = counts_s[Q]
                p0 = pos_s[Q] - cnt

                def serve(pending):
                    def scan_seg(j, pend):
                        @pl.when(pend > RING - L)
                        def _():
                            def drain1(_i, c):
                                pltpu.make_async_copy(
                                    stage_hbm.at[pl.ds(0, RP)],
                                    ring_v.at[pl.ds(0, RP)],
                                    semr).wait()
                                return c

                            lax.fori_loop(0, pend, drain1, 0)

                        pend = jnp.where(pend > RING - L, 0, pend)
                        vv = lst2[pl.ds(p0 + j * L, L)]
                        act = (lanes + j * L) < cnt
                        act32 = act.astype(jnp.int32)
                        for lane in range(L):
                            mbit = act32[lane]
                            slot = pend

                            @pl.when(mbit == 1)
                            def _():
                                q = vv[lane] & 127
                                b = vv[lane] >> 15
                                qs = jnp.full((L,), q, jnp.int32)
                                dbase = jnp.full((L,), boff, jnp.int32)
                                for jj in range(D // L):
                                    dv = dbase + lanes + jj * L
                                    vals = plsc.load_gather(qbuf, [dv, qs])
                                    ring_v[pl.ds(slot * RP + jj * L, L)] = vals
                                pltpu.async_copy(
                                    ring_v.at[pl.ds(slot * RP, RP)],
                                    stage_hbm.at[pl.ds(b * RP, RP)],
                                    semr)

                            pend = pend + mbit
                        return pend

                    return lax.fori_loop(0, (cnt + L - 1) // L, scan_seg,
                                         pending)

                return lax.cond((g < NCHUNK) & (cnt > 0), serve,
                                lambda p: p, pending)

            for r in range(NR - 1):
                issue(r, semq[r], r * D)

            def group(q8, pending):
                qb = q8 * NR
                for r in range(NR):
                    Q = qb + r
                    wait_q(Q, semq[r], r * D)
                    rn = (r + NR - 1) % NR
                    issue(Q + NR - 1, semq[rn], rn * D)
                    pending = process(Q, r * D, pending)
                return pending

            pending = lax.fori_loop(0, NQ // NR, group, 0)

            def final_drain(_i, carry):
                pltpu.make_async_copy(
                    stage_hbm.at[pl.ds(0, RP)], ring_v.at[pl.ds(0, RP)],
                    semr).wait()
                return carry

            lax.fori_loop(0, pending, final_drain, 0)

        one_table(uids_hbm, utT_hbm, ustage_hbm)
        one_table(iids_hbm, itT_hbm, istage_hbm)

    return kern


@functools.cache
def _make_dot_kernel(B):
    mesh = plsc.VectorSubcoreMesh(core_axis_name="c", subcore_axis_name="s")
    BPW = B // NW
    GRP = 256

    @functools.partial(
        pl.kernel,
        mesh=mesh,
        out_type=jax.ShapeDtypeStruct((B,), jnp.float32),
        compiler_params=pltpu.CompilerParams(
            needs_layout_passes=False, use_tc_tiling_on_sc=False),
        scratch_types=[
            pltpu.VMEM((GRP * RP,), jnp.float32),
            pltpu.VMEM((GRP * RP,), jnp.float32),
            pltpu.VMEM((BPW,), jnp.int32),
            pltpu.VMEM((BPW,), jnp.int32),
            pltpu.VMEM((D * D,), jnp.float32),
            pltpu.VMEM((D * D,), jnp.float32),
            pltpu.VMEM((BPW,), jnp.float32),
        ],
    )
    def kern(uids_hbm, iids_hbm, ustage_hbm, istage_hbm, utail_hbm, itail_hbm,
             out_hbm, uv, iv, uidv, iidv, utl, itl, out_v):
        wid = lax.axis_index("s") * NC + lax.axis_index("c")
        base = wid * BPW
        lanes = lax.iota(jnp.int32, L)

        pltpu.sync_copy(uids_hbm.at[pl.ds(base, BPW)], uidv)
        pltpu.sync_copy(iids_hbm.at[pl.ds(base, BPW)], iidv)
        pltpu.sync_copy(utail_hbm, utl)
        pltpu.sync_copy(itail_hbm, itl)

        for grp in range(BPW // GRP):
            g0 = base + grp * GRP
            pltpu.sync_copy(ustage_hbm.at[pl.ds(g0 * RP, GRP * RP)], uv)
            pltpu.sync_copy(istage_hbm.at[pl.ds(g0 * RP, GRP * RP)], iv)

            def body(it, carry):
                r0 = it * L
                res = jnp.zeros((L,), jnp.float32)
                for kk in range(L):
                    r = r0 + kk
                    acc = jnp.zeros((L,), jnp.float32)
                    for j in range(D // L):
                        pu = uv[pl.ds(r * RP + j * L, L)]
                        pi = iv[pl.ds(r * RP + j * L, L)]
                        acc = acc + pu * pi
                    res = jnp.where(lanes == kk, jnp.sum(acc), res)

                ob = grp * GRP + r0
                uidvec = uidv[pl.ds(ob, L)]
                iidvec = iidv[pl.ds(ob, L)]
                tl = (uidvec >= TAIL) | (iidvec >= TAIL)
                tl32 = tl.astype(jnp.int32)
                nt = plsc.all_reduce_population_count(tl)[0]

                @pl.when(nt > 0)
                def _():
                    fixed = res
                    for lane in range(L):
                        @pl.when(tl32[lane] == 1)
                        def _():
                            r = r0 + lane
                            uid = uidvec[lane]
                            iid = iidvec[lane]
                            uo = lax.max(uid - TAIL, 0) * D
                            io = lax.max(iid - TAIL, 0) * D
                            acc2 = jnp.zeros((L,), jnp.float32)
                            for j in range(D // L):
                                us = uv[pl.ds(r * RP + j * L, L)]
                                ut = utl[pl.ds(uo + j * L, L)]
                                uu = jnp.where(uid >= TAIL, ut, us)
                                ss = iv[pl.ds(r * RP + j * L, L)]
                                st = itl[pl.ds(io + j * L, L)]
                                ii = jnp.where(iid >= TAIL, st, ss)
                                acc2 = acc2 + uu * ii
                            s = jnp.sum(acc2)
                            cur = out_v[pl.ds(ob, L)]
                            out_v[pl.ds(ob, L)] = jnp.where(
                                lanes == lane, s, cur)

                    out_v[pl.ds(ob, L)] = jnp.where(
                        tl, out_v[pl.ds(ob, L)], fixed)

                @pl.when(nt == 0)
                def _():
                    out_v[pl.ds(ob, L)] = res

                return carry

            lax.fori_loop(0, GRP // L, body, 0)

        pltpu.sync_copy(out_v, out_hbm.at[pl.ds(base, BPW)])

    return kern


def kernel(user_ids, item_ids, user_table, item_table):
    B = user_ids.shape[0]
    V = user_table.shape[0]
    uids = user_ids.astype(jnp.int32)
    iids = item_ids.astype(jnp.int32)
    utail = user_table[TAIL:, :].reshape(-1)
    itail = item_table[TAIL:, :].reshape(-1)
    gather = _make_gather_kernel(B, V)
    ustage, istage = gather(uids, iids, user_table.T, item_table.T)
    dot = _make_dot_kernel(B)
    return dot(uids, iids, ustage, istage, utail, itail)


# speedup vs baseline: 3.1409x; 1.4761x over previous
import functools

import jax
import jax.numpy as jnp
from jax import lax
from jax.experimental import pallas as pl
from jax.experimental.pallas import tpu as pltpu
from jax.experimental.pallas import tpu_sc as plsc

D = 64
L = 16
NC = 2
NS = 16
NW = NC * NS
CW = 512
NCHUNK = 999936 // CW
TAIL = 999936
RING = 64
RP = 128


NQ = 124
NR = 4
UW = 256


@functools.cache
def _make_gather_kernel(B, V):
    mesh = plsc.VectorSubcoreMesh(core_axis_name="c", subcore_axis_name="s")
    NV = B // L

    @functools.partial(
        pl.kernel,
        mesh=mesh,
        out_type=(jax.ShapeDtypeStruct((B * RP,), jnp.float32),
                  jax.ShapeDtypeStruct((B * RP,), jnp.float32)),
        compiler_params=pltpu.CompilerParams(
            needs_layout_passes=False, use_tc_tiling_on_sc=True),
        scratch_types=[
            pltpu.VMEM((B,), jnp.int32),
            pltpu.VMEM((B + L,), jnp.int32),
            pltpu.VMEM((B + L,), jnp.int32),
            pltpu.VMEM((NR * D, UW), jnp.float32),
            pltpu.VMEM((RING * RP,), jnp.float32),
            pltpu.SMEM((128,), jnp.int32),
            pltpu.SMEM((128,), jnp.int32),
        ] + [pltpu.SemaphoreType.DMA] * (NR + 1),
    )
    def kern(uids_hbm, iids_hbm, utT_hbm, itT_hbm, ustage_hbm, istage_hbm,
             ids_v, lst, lst2, qbuf, ring_v, counts_s, pos_s, *sems):
        semq = sems[:NR]
        semr = sems[NR]
        wid = lax.axis_index("s") * NC + lax.axis_index("c")
        lanes = lax.iota(jnp.int32, L)

        def one_table(ids_hbm, tT_hbm, stage_hbm):
            pltpu.sync_copy(ids_hbm, ids_v)

            def scan_ids(j, off):
                idv = ids_v[pl.ds(j * L, L)]
                bv = jnp.full((L,), j * L, jnp.int32) + lanes
                keep = ((idv >> 9) & (NW - 1)) == wid
                cnt = plsc.all_reduce_population_count(keep)[0]
                packed = (bv << 15) | ((idv >> 14) << 9) | (idv & 0x1FF)
                plsc.store_compressed(lst.at[pl.ds(off, L)], packed, mask=keep)
                return off + cnt

            n = lax.fori_loop(0, NV, scan_ids, 0)
            nv = (n + L - 1) // L

            def zero(i, c):
                counts_s[i] = 0
                return c

            lax.fori_loop(0, 128, zero, 0)

            def hist(j, c):
                vv = lst[pl.ds(j * L, L)]
                for lane in range(L):
                    kk = (vv[lane] >> 8) & 0x7F
                    inc = jnp.where(j * L + lane < n, 1, 0)
                    counts_s[kk] = counts_s[kk] + inc
                return c

            lax.fori_loop(0, nv, hist, 0)

            def pref(k, run):
                pos_s[k] = run
                return run + counts_s[k]

            lax.fori_loop(0, 128, pref, 0)

            def scat(j, c):
                vv = lst[pl.ds(j * L, L)]
                for lane in range(L):
                    kk = (vv[lane] >> 8) & 0x7F
                    p = pos_s[kk]
                    ok = j * L + lane < n
                    onehot = (lanes == lane) & ok
                    plsc.store_scatter(
                        lst2, [jnp.full((L,), p, jnp.int32)], vv, mask=onehot)
                    pos_s[kk] = p + jnp.where(ok, 1, 0)
                return c

            lax.fori_loop(0, nv, scat, 0)

            def issue(Q, sem, boff):
                g = wid + (Q >> 1) * NW
                col = g * CW + (Q & 1) * UW

                @pl.when(g < NCHUNK)
                def _():
                    pltpu.async_copy(
                        tT_hbm.at[:, pl.ds(col, UW)],
                        qbuf.at[pl.ds(boff, D), :], sem)

            def wait_q(Q, sem, boff):
                g = wid + (Q >> 1) * NW
                col = g * CW + (Q & 1) * UW

                @pl.when(g < NCHUNK)
                def _():
                    pltpu.make_async_copy(
                        tT_hbm.at[:, pl.ds(col, UW)],
                        qbuf.at[pl.ds(boff, D), :], sem).wait()

            def process(Q, boff, pending):
                g = wid + (Q >> 1) * NW
                cnt = counts_s[Q]
                p0 = pos_s[Q] - cnt

                def serve(pending):
                    def scan_seg(j, pend):
                        @pl.when(pend > RING - L)
                        def _():
                            def drain1(_i, c):
                                pltpu.make_async_copy(
                                    stage_hbm.at[pl.ds(0, RP)],
                                    ring_v.at[pl.ds(0, RP)],
                                    semr).wait()
                                return c

                            lax.fori_loop(0, pend, drain1, 0)

                        pend = jnp.where(pend > RING - L, 0, pend)
                        vv = lst2[pl.ds(p0 + j * L, L)]
                        act = (lanes + j * L) < cnt
                        act32 = act.astype(jnp.int32)
                        for lane in range(L):
                            mbit = act32[lane]
                            slot = pend

                            @pl.when(mbit == 1)
                            def _():
                                q = vv[lane] & 255
                                b = vv[lane] >> 15
                                qs = jnp.full((L,), q, jnp.int32)
                                dbase = jnp.full((L,), boff, jnp.int32)
                                for jj in range(D // L):
                                    dv = dbase + lanes + jj * L
                                    vals = plsc.load_gather(qbuf, [dv, qs])
                                    ring_v[pl.ds(slot * RP + jj * L, L)] = vals
                                pltpu.async_copy(
                                    ring_v.at[pl.ds(slot * RP, RP)],
                                    stage_hbm.at[pl.ds(b * RP, RP)],
                                    semr)

                            pend = pend + mbit
                        return pend

                    return lax.fori_loop(0, (cnt + L - 1) // L, scan_seg,
                                         pending)

                return lax.cond((g < NCHUNK) & (cnt > 0), serve,
                                lambda p: p, pending)

            for r in range(NR - 1):
                issue(r, semq[r], r * D)

            def group(q8, pending):
                qb = q8 * NR
                for r in range(NR):
                    Q = qb + r
                    wait_q(Q, semq[r], r * D)
                    rn = (r + NR - 1) % NR
                    issue(Q + NR - 1, semq[rn], rn * D)
                    pending = process(Q, r * D, pending)
                return pending

            pending = lax.fori_loop(0, NQ // NR, group, 0)

            def final_drain(_i, carry):
                pltpu.make_async_copy(
                    stage_hbm.at[pl.ds(0, RP)], ring_v.at[pl.ds(0, RP)],
                    semr).wait()
                return carry

            lax.fori_loop(0, pending, final_drain, 0)

        one_table(uids_hbm, utT_hbm, ustage_hbm)
        one_table(iids_hbm, itT_hbm, istage_hbm)

    return kern


@functools.cache
def _make_dot_kernel(B):
    mesh = plsc.VectorSubcoreMesh(core_axis_name="c", subcore_axis_name="s")
    BPW = B // NW
    GRP = 256

    @functools.partial(
        pl.kernel,
        mesh=mesh,
        out_type=jax.ShapeDtypeStruct((B,), jnp.float32),
        compiler_params=pltpu.CompilerParams(
            needs_layout_passes=False, use_tc_tiling_on_sc=False),
        scratch_types=[
            pltpu.VMEM((GRP * RP,), jnp.float32),
            pltpu.VMEM((GRP * RP,), jnp.float32),
            pltpu.VMEM((BPW,), jnp.int32),
            pltpu.VMEM((BPW,), jnp.int32),
            pltpu.VMEM((D * D,), jnp.float32),
            pltpu.VMEM((D * D,), jnp.float32),
            pltpu.VMEM((BPW,), jnp.float32),
        ],
    )
    def kern(uids_hbm, iids_hbm, ustage_hbm, istage_hbm, utail_hbm, itail_hbm,
             out_hbm, uv, iv, uidv, iidv, utl, itl, out_v):
        wid = lax.axis_index("s") * NC + lax.axis_index("c")
        base = wid * BPW
        lanes = lax.iota(jnp.int32, L)

        pltpu.sync_copy(uids_hbm.at[pl.ds(base, BPW)], uidv)
        pltpu.sync_copy(iids_hbm.at[pl.ds(base, BPW)], iidv)
        pltpu.sync_copy(utail_hbm, utl)
        pltpu.sync_copy(itail_hbm, itl)

        for grp in range(BPW // GRP):
            g0 = base + grp * GRP
            pltpu.sync_copy(ustage_hbm.at[pl.ds(g0 * RP, GRP * RP)], uv)
            pltpu.sync_copy(istage_hbm.at[pl.ds(g0 * RP, GRP * RP)], iv)

            def body(it, carry):
                r0 = it * L
                res = jnp.zeros((L,), jnp.float32)
                for kk in range(L):
                    r = r0 + kk
                    acc = jnp.zeros((L,), jnp.float32)
                    for j in range(D // L):
                        pu = uv[pl.ds(r * RP + j * L, L)]
                        pi = iv[pl.ds(r * RP + j * L, L)]
                        acc = acc + pu * pi
                    res = jnp.where(lanes == kk, jnp.sum(acc), res)

                ob = grp * GRP + r0
                uidvec = uidv[pl.ds(ob, L)]
                iidvec = iidv[pl.ds(ob, L)]
                tl = (uidvec >= TAIL) | (iidvec >= TAIL)
                tl32 = tl.astype(jnp.int32)
                nt = plsc.all_reduce_population_count(tl)[0]

                @pl.when(nt > 0)
                def _():
                    fixed = res
                    for lane in range(L):
                        @pl.when(tl32[lane] == 1)
                        def _():
                            r = r0 + lane
                            uid = uidvec[lane]
                            iid = iidvec[lane]
                            uo = lax.max(uid - TAIL, 0) * D
                            io = lax.max(iid - TAIL, 0) * D
                            acc2 = jnp.zeros((L,), jnp.float32)
                            for j in range(D // L):
                                us = uv[pl.ds(r * RP + j * L, L)]
                                ut = utl[pl.ds(uo + j * L, L)]
                                uu = jnp.where(uid >= TAIL, ut, us)
                                ss = iv[pl.ds(r * RP + j * L, L)]
                                st = itl[pl.ds(io + j * L, L)]
                                ii = jnp.where(iid >= TAIL, st, ss)
                                acc2 = acc2 + uu * ii
                            s = jnp.sum(acc2)
                            cur = out_v[pl.ds(ob, L)]
                            out_v[pl.ds(ob, L)] = jnp.where(
                                lanes == lane, s, cur)

                    out_v[pl.ds(ob, L)] = jnp.where(
                        tl, out_v[pl.ds(ob, L)], fixed)

                @pl.when(nt == 0)
                def _():
                    out_v[pl.ds(ob, L)] = res

                return carry

            lax.fori_loop(0, GRP // L, body, 0)

        pltpu.sync_copy(out_v, out_hbm.at[pl.ds(base, BPW)])

    return kern


def kernel(user_ids, item_ids, user_table, item_table):
    B = user_ids.shape[0]
    V = user_table.shape[0]
    uids = user_ids.astype(jnp.int32)
    iids = item_ids.astype(jnp.int32)
    utail = user_table[TAIL:, :].reshape(-1)
    itail = item_table[TAIL:, :].reshape(-1)
    gather = _make_gather_kernel(B, V)
    ustage, istage = gather(uids, iids, user_table.T, item_table.T)
    dot = _make_dot_kernel(B)
    return dot(uids, iids, ustage, istage, utail, itail)


# speedup vs baseline: 3.3975x; 1.0817x over previous
import functools

import jax
import jax.numpy as jnp
from jax import lax
from jax.experimental import pallas as pl
from jax.experimental.pallas import tpu as pltpu
from jax.experimental.pallas import tpu_sc as plsc

D = 64
L = 16
NC = 2
NS = 16
NW = NC * NS
CW = 512
NCHUNK = 999936 // CW
TAIL = 999936
RING = 64
RP = 128


NQ = 62
NR = 2
UW = 512


@functools.cache
def _make_gather_kernel(B, V):
    mesh = plsc.VectorSubcoreMesh(core_axis_name="c", subcore_axis_name="s")
    NV = B // L

    @functools.partial(
        pl.kernel,
        mesh=mesh,
        out_type=(jax.ShapeDtypeStruct((B * RP,), jnp.float32),
                  jax.ShapeDtypeStruct((B * RP,), jnp.float32)),
        compiler_params=pltpu.CompilerParams(
            needs_layout_passes=False, use_tc_tiling_on_sc=True),
        scratch_types=[
            pltpu.VMEM((B,), jnp.int32),
            pltpu.VMEM((B + L,), jnp.int32),
            pltpu.VMEM((B + L,), jnp.int32),
            pltpu.VMEM((NR * D, UW), jnp.float32),
            pltpu.VMEM((RING * RP,), jnp.float32),
            pltpu.SMEM((64,), jnp.int32),
            pltpu.SMEM((64,), jnp.int32),
        ] + [pltpu.SemaphoreType.DMA] * (NR + 1),
    )
    def kern(uids_hbm, iids_hbm, utT_hbm, itT_hbm, ustage_hbm, istage_hbm,
             ids_v, lst, lst2, qbuf, ring_v, counts_s, pos_s, *sems):
        semq = sems[:NR]
        semr = sems[NR]
        wid = lax.axis_index("s") * NC + lax.axis_index("c")
        lanes = lax.iota(jnp.int32, L)

        def one_table(ids_hbm, tT_hbm, stage_hbm):
            pltpu.sync_copy(ids_hbm, ids_v)

            def scan_ids(j, off):
                idv = ids_v[pl.ds(j * L, L)]
                bv = jnp.full((L,), j * L, jnp.int32) + lanes
                keep = ((idv >> 9) & (NW - 1)) == wid
                cnt = plsc.all_reduce_population_count(keep)[0]
                packed = (bv << 15) | ((idv >> 14) << 9) | (idv & 0x1FF)
                plsc.store_compressed(lst.at[pl.ds(off, L)], packed, mask=keep)
                return off + cnt

            n = lax.fori_loop(0, NV, scan_ids, 0)
            nv = (n + L - 1) // L

            def zero(i, c):
                counts_s[i] = 0
                return c

            lax.fori_loop(0, 64, zero, 0)

            def hist(j, c):
                vv = lst[pl.ds(j * L, L)]
                for lane in range(L):
                    kk = (vv[lane] >> 9) & 0x3F
                    inc = jnp.where(j * L + lane < n, 1, 0)
                    counts_s[kk] = counts_s[kk] + inc
                return c

            lax.fori_loop(0, nv, hist, 0)

            def pref(k, run):
                pos_s[k] = run
                return run + counts_s[k]

            lax.fori_loop(0, 64, pref, 0)

            def scat(j, c):
                vv = lst[pl.ds(j * L, L)]
                for lane in range(L):
                    kk = (vv[lane] >> 9) & 0x3F
                    p = pos_s[kk]
                    ok = j * L + lane < n
                    onehot = (lanes == lane) & ok
                    plsc.store_scatter(
                        lst2, [jnp.full((L,), p, jnp.int32)], vv, mask=onehot)
                    pos_s[kk] = p + jnp.where(ok, 1, 0)
                return c

            lax.fori_loop(0, nv, scat, 0)

            def issue(Q, sem, boff):
                g = wid + Q * NW
                col = g * CW

                @pl.when(g < NCHUNK)
                def _():
                    pltpu.async_copy(
                        tT_hbm.at[:, pl.ds(col, UW)],
                        qbuf.at[pl.ds(boff, D), :], sem)

            def wait_q(Q, sem, boff):
                g = wid + Q * NW
                col = g * CW

                @pl.when(g < NCHUNK)
                def _():
                    pltpu.make_async_copy(
                        tT_hbm.at[:, pl.ds(col, UW)],
                        qbuf.at[pl.ds(boff, D), :], sem).wait()

            def process(Q, boff, pending):
                g = wid + Q * NW
                cnt = counts_s[Q]
                p0 = pos_s[Q] - cnt

                def serve(pending):
                    def scan_seg(j, pend):
                        @pl.when(pend > RING - L)
                        def _():
                            def drain1(_i, c):
                                pltpu.make_async_copy(
                                    stage_hbm.at[pl.ds(0, RP)],
                                    ring_v.at[pl.ds(0, RP)],
                                    semr).wait()
                                return c

                            lax.fori_loop(0, pend, drain1, 0)

                        pend = jnp.where(pend > RING - L, 0, pend)
                        vv = lst2[pl.ds(p0 + j * L, L)]
                        act = (lanes + j * L) < cnt
                        act32 = act.astype(jnp.int32)
                        for lane in range(L):
                            mbit = act32[lane]
                            slot = pend

                            @pl.when(mbit == 1)
                            def _():
                                q = vv[lane] & 511
                                b = vv[lane] >> 15
                                qs = jnp.full((L,), q, jnp.int32)
                                dbase = jnp.full((L,), boff, jnp.int32)
                                for jj in range(D // L):
                                    dv = dbase + lanes + jj * L
                                    vals = plsc.load_gather(qbuf, [dv, qs])
                                    ring_v[pl.ds(slot * RP + jj * L, L)] = vals
                                pltpu.async_copy(
                                    ring_v.at[pl.ds(slot * RP, RP)],
                                    stage_hbm.at[pl.ds(b * RP, RP)],
                                    semr)

                            pend = pend + mbit
                        return pend

                    return lax.fori_loop(0, (cnt + L - 1) // L, scan_seg,
                                         pending)

                return lax.cond((g < NCHUNK) & (cnt > 0), serve,
                                lambda p: p, pending)

            for r in range(NR - 1):
                issue(r, semq[r], r * D)

            def group(q8, pending):
                qb = q8 * NR
                for r in range(NR):
                    Q = qb + r
                    wait_q(Q, semq[r], r * D)
                    rn = (r + NR - 1) % NR
                    issue(Q + NR - 1, semq[rn], rn * D)
                    pending = process(Q, r * D, pending)
                return pending

            pending = lax.fori_loop(0, NQ // NR, group, 0)

            def final_drain(_i, carry):
                pltpu.make_async_copy(
                    stage_hbm.at[pl.ds(0, RP)], ring_v.at[pl.ds(0, RP)],
                    semr).wait()
                return carry

            lax.fori_loop(0, pending, final_drain, 0)

        one_table(uids_hbm, utT_hbm, ustage_hbm)
        one_table(iids_hbm, itT_hbm, istage_hbm)

    return kern


@functools.cache
def _make_dot_kernel(B):
    mesh = plsc.VectorSubcoreMesh(core_axis_name="c", subcore_axis_name="s")
    BPW = B // NW
    GRP = 256

    @functools.partial(
        pl.kernel,
        mesh=mesh,
        out_type=jax.ShapeDtypeStruct((B,), jnp.float32),
        compiler_params=pltpu.CompilerParams(
            needs_layout_passes=False, use_tc_tiling_on_sc=False),
        scratch_types=[
            pltpu.VMEM((GRP * RP,), jnp.float32),
            pltpu.VMEM((GRP * RP,), jnp.float32),
            pltpu.VMEM((BPW,), jnp.int32),
            pltpu.VMEM((BPW,), jnp.int32),
            pltpu.VMEM((D * D,), jnp.float32),
            pltpu.VMEM((D * D,), jnp.float32),
            pltpu.VMEM((BPW,), jnp.float32),
        ],
    )
    def kern(uids_hbm, iids_hbm, ustage_hbm, istage_hbm, utail_hbm, itail_hbm,
             out_hbm, uv, iv, uidv, iidv, utl, itl, out_v):
        wid = lax.axis_index("s") * NC + lax.axis_index("c")
        base = wid * BPW
        lanes = lax.iota(jnp.int32, L)

        pltpu.sync_copy(uids_hbm.at[pl.ds(base, BPW)], uidv)
        pltpu.sync_copy(iids_hbm.at[pl.ds(base, BPW)], iidv)
        pltpu.sync_copy(utail_hbm, utl)
        pltpu.sync_copy(itail_hbm, itl)

        for grp in range(BPW // GRP):
            g0 = base + grp * GRP
            pltpu.sync_copy(ustage_hbm.at[pl.ds(g0 * RP, GRP * RP)], uv)
            pltpu.sync_copy(istage_hbm.at[pl.ds(g0 * RP, GRP * RP)], iv)

            def body(it, carry):
                r0 = it * L
                res = jnp.zeros((L,), jnp.float32)
                for kk in range(L):
                    r = r0 + kk
                    acc = jnp.zeros((L,), jnp.float32)
                    for j in range(D // L):
                        pu = uv[pl.ds(r * RP + j * L, L)]
                        pi = iv[pl.ds(r * RP + j * L, L)]
                        acc = acc + pu * pi
                    res = jnp.where(lanes == kk, jnp.sum(acc), res)

                ob = grp * GRP + r0
                uidvec = uidv[pl.ds(ob, L)]
                iidvec = iidv[pl.ds(ob, L)]
                tl = (uidvec >= TAIL) | (iidvec >= TAIL)
                tl32 = tl.astype(jnp.int32)
                nt = plsc.all_reduce_population_count(tl)[0]

                @pl.when(nt > 0)
                def _():
                    fixed = res
                    for lane in range(L):
                        @pl.when(tl32[lane] == 1)
                        def _():
                            r = r0 + lane
                            uid = uidvec[lane]
                            iid = iidvec[lane]
                            uo = lax.max(uid - TAIL, 0) * D
                            io = lax.max(iid - TAIL, 0) * D
                            acc2 = jnp.zeros((L,), jnp.float32)
                            for j in range(D // L):
                                us = uv[pl.ds(r * RP + j * L, L)]
                                ut = utl[pl.ds(uo + j * L, L)]
                                uu = jnp.where(uid >= TAIL, ut, us)
                                ss = iv[pl.ds(r * RP + j * L, L)]
                                st = itl[pl.ds(io + j * L, L)]
                                ii = jnp.where(iid >= TAIL, st, ss)
                                acc2 = acc2 + uu * ii
                            s = jnp.sum(acc2)
                            cur = out_v[pl.ds(ob, L)]
                            out_v[pl.ds(ob, L)] = jnp.where(
                                lanes == lane, s, cur)

                    out_v[pl.ds(ob, L)] = jnp.where(
                        tl, out_v[pl.ds(ob, L)], fixed)

                @pl.when(nt == 0)
                def _():
                    out_v[pl.ds(ob, L)] = res

                return carry

            lax.fori_loop(0, GRP // L, body, 0)

        pltpu.sync_copy(out_v, out_hbm.at[pl.ds(base, BPW)])

    return kern


def kernel(user_ids, item_ids, user_table, item_table):
    B = user_ids.shape[0]
    V = user_table.shape[0]
    uids = user_ids.astype(jnp.int32)
    iids = item_ids.astype(jnp.int32)
    utail = user_table[TAIL:, :].reshape(-1)
    itail = item_table[TAIL:, :].reshape(-1)
    gather = _make_gather_kernel(B, V)
    ustage, istage = gather(uids, iids, user_table.T, item_table.T)
    dot = _make_dot_kernel(B)
    return dot(uids, iids, ustage, istage, utail, itail)
